# Initial kernel scaffold; baseline (speedup 1.0000x reference)
import jax
import jax.numpy as jnp
from jax.experimental import pallas as pl


def kernel(x, key):
    raise NotImplementedError("write your pallas kernel here")


# speedup vs baseline: 1.1596x; 1.1596x over previous
import jax
import jax.numpy as jnp
from jax.experimental import pallas as pl
from jax.experimental.pallas import tpu as pltpu

_RATIO = 0.2


def _cut_bounds(key, h, w):
    cut_x = int(w * _RATIO + 0.5)
    cut_y = int(h * _RATIO + 0.5)
    k1, k2 = jax.random.split(key)
    offset_x = jax.random.randint(k1, (1, 1), 0, w + (1 - cut_x % 2))[0, 0]
    offset_y = jax.random.randint(k2, (1, 1), 0, h + (1 - cut_y % 2))[0, 0]
    x0 = jnp.clip(offset_x - cut_x // 2, 0, w - 1)
    x1 = jnp.clip(offset_x - cut_x // 2 + cut_x - 1, 0, w - 1)
    y0 = jnp.clip(offset_y - cut_y // 2, 0, h - 1)
    y1 = jnp.clip(offset_y - cut_y // 2 + cut_y - 1, 0, h - 1)
    return jnp.stack([y0, y1, x0, x1]).astype(jnp.int32)


def _body(b_ref, x_ref, o_ref):
    by, w, _ = x_ref.shape
    i = pl.program_id(0)
    rows = i * by + jax.lax.broadcasted_iota(jnp.int32, (by, w), 0)
    cols = jax.lax.broadcasted_iota(jnp.int32, (by, w), 1)
    inside = (rows >= b_ref[0]) & (rows <= b_ref[1]) \
        & (cols >= b_ref[2]) & (cols <= b_ref[3])
    mask = jnp.where(inside, 0.0, 1.0).astype(o_ref.dtype)
    o_ref[...] = x_ref[...] * mask[:, :, None]


def kernel(x, key):
    h, w, c = x.shape
    bounds = _cut_bounds(key, h, w)
    by = 8
    return pl.pallas_call(
        _body,
        grid=(h // by,),
        in_specs=[
            pl.BlockSpec(memory_space=pltpu.SMEM),
            pl.BlockSpec((by, w, c), lambda i: (i, 0, 0)),
        ],
        out_specs=pl.BlockSpec((by, w, c), lambda i: (i, 0, 0)),
        out_shape=jax.ShapeDtypeStruct((h, w, c), x.dtype),
    )(bounds, x)


# speedup vs baseline: 1.1674x; 1.0067x over previous
import jax
import jax.numpy as jnp
from jax.experimental import pallas as pl
from jax.experimental.pallas import tpu as pltpu

_RATIO = 0.2


def _cut_bounds(key, h, w):
    cut_x = int(w * _RATIO + 0.5)
    cut_y = int(h * _RATIO + 0.5)
    k1, k2 = jax.random.split(key)
    offset_x = jax.random.randint(k1, (1, 1), 0, w + (1 - cut_x % 2))[0, 0]
    offset_y = jax.random.randint(k2, (1, 1), 0, h + (1 - cut_y % 2))[0, 0]
    x0 = jnp.clip(offset_x - cut_x // 2, 0, w - 1)
    x1 = jnp.clip(offset_x - cut_x // 2 + cut_x - 1, 0, w - 1)
    y0 = jnp.clip(offset_y - cut_y // 2, 0, h - 1)
    y1 = jnp.clip(offset_y - cut_y // 2 + cut_y - 1, 0, h - 1)
    return jnp.stack([y0, y1, x0, x1]).astype(jnp.int32)


def _body(b_ref, x_ref, o_ref):
    by, w, _ = x_ref.shape
    i = pl.program_id(0)
    rows = i * by + jax.lax.broadcasted_iota(jnp.int32, (by, w), 0)
    cols = jax.lax.broadcasted_iota(jnp.int32, (by, w), 1)
    inside = (rows >= b_ref[0]) & (rows <= b_ref[1]) \
        & (cols >= b_ref[2]) & (cols <= b_ref[3])
    mask = jnp.where(inside, 0.0, 1.0).astype(o_ref.dtype)
    o_ref[...] = x_ref[...] * mask[:, :, None]


def kernel(x, key):
    h, w, c = x.shape
    bounds = _cut_bounds(key, h, w)
    by = 16
    return pl.pallas_call(
        _body,
        grid=(h // by,),
        in_specs=[
            pl.BlockSpec(memory_space=pltpu.SMEM),
            pl.BlockSpec((by, w, c), lambda i: (i, 0, 0)),
        ],
        out_specs=pl.BlockSpec((by, w, c), lambda i: (i, 0, 0)),
        out_shape=jax.ShapeDtypeStruct((h, w, c), x.dtype),
    )(bounds, x)
